# baseline (speedup 1.0000x reference)
import functools

import jax
import jax.numpy as jnp
from jax import lax
from jax.experimental import pallas as pl
from jax.experimental.pallas import tpu as pltpu
from jax.experimental.pallas import tpu_sc as plsc

B, T, D = 8, 512, 768
NM = 8
NR_T = 4
IMG_V = 576
NR_I = 144
P = B * T
NW = 32
C = P // NW
NG = C // 16
IMG_PITCH = 152


def _sc_temporal_body(noise_t_ref, t0_ref, v1, v2, v3, v4, v5, v6, v7, v8,
                      out_ref, nbuf, srcb, dstb,
                      gbufa, gbufb, gbufc, gbufd, gbufe, gbuff,
                      sem_g, sem_s):
    vals = (v1, v2, v3, v4, v5, v6, v7, v8)
    wid = lax.axis_index("s") * 2 + lax.axis_index("c")
    base = wid * C
    pltpu.sync_copy(noise_t_ref.at[pl.ds(base * NM, C * NM)], nbuf)
    iota16 = lax.broadcasted_iota(jnp.int32, (16,), 0)
    LROW = C + 16
    offs = [0] * NM
    last_pair = [jnp.full((16,), -1, jnp.int32) for _ in range(NM)]
    for g in range(NG):
        lane_base = (g * 16 + iota16) * NM
        nvs = [plsc.load_gather(nbuf, [lane_base + m]) for m in range(NM)]
        pos = base + g * 16 + iota16
        for m in range(NM):
            r = jnp.zeros((16,), jnp.int32)
            for n in range(NM):
                if n == m:
                    continue
                cond = nvs[n] < nvs[m]
                if n < m:
                    cond = cond | (nvs[n] == nvs[m])
                r = r + cond.astype(jnp.int32)
            mask = r < NR_T
            b_vec = pos // T
            t_vec = pos - b_vec * T
            dst = (b_vec * (NR_T + 1) + 1 + r) * T + t_vec
            cs = plsc.cumsum(mask.astype(jnp.int32))
            slot = m * LROW + offs[m] + cs - 1
            plsc.store_scatter(srcb, [slot], pos, mask=mask)
            plsc.store_scatter(dstb, [slot], dst, mask=mask)
            offs[m] = offs[m] + jnp.max(
                plsc.all_reduce_population_count(mask))
            last_pair[m] = jnp.maximum(
                last_pair[m], jnp.where(mask, pos * 32768 + dst, -1))
    zero16 = jnp.zeros((16,), jnp.int32)
    for m in range(NM):
        tail = m * LROW + offs[m] + iota16
        best = jnp.max(last_pair[m])
        ls = best // 32768
        plsc.store_scatter(srcb, [tail], zero16 + ls, mask=iota16 >= 0)
        plsc.store_scatter(dstb, [tail], zero16 + (best - ls * 32768),
                           mask=iota16 >= 0)
    bufs = (gbufa, gbufb, gbufc, gbufd, gbufe, gbuff)

    def wait_bytes(sem):
        pltpu.make_async_copy(t0_ref.at[pl.ds(0, 16)], gbufa, sem).wait()

    def gather_blk(m, i, buf):
        sidx = srcb[pl.ds(m * LROW + i * 16, 16)]
        pltpu.async_copy(vals[m].at[sidx], buf, sem_g)

    def scatter_blk(m, i, buf):
        didx = dstb[pl.ds(m * LROW + i * 16, 16)]
        pltpu.async_copy(buf, out_ref.at[didx], sem_s)

    nbs = [(offs[m] + 15) // 16 for m in range(NM)]
    for k in range(3):
        @pl.when(nbs[0] > k)
        def _(k=k):
            gather_blk(0, k, bufs[k])
    total = 0
    for m in range(NM):
        nb = nbs[m]

        def it(i, carry, m=m, goff=total, nb=nb):
            g = goff + i
            @pl.when(g >= 3)
            def _():
                wait_bytes(sem_s)
            for p in range(6):
                @pl.when(((g + 3) % 6 == p) & (i + 3 < nb))
                def _(p=p):
                    gather_blk(m, i + 3, bufs[p])
            wait_bytes(sem_g)
            for p in range(6):
                @pl.when(g % 6 == p)
                def _(p=p):
                    scatter_blk(m, i, bufs[p])
            return carry

        lax.fori_loop(0, nb, it, 0)
        total = total + nb
        if m + 1 < NM:
            for k in range(3):
                for p in range(6):
                    @pl.when((nbs[m + 1] > k) & ((total + k) % 6 == p))
                    def _(k=k, p=p, m2=m + 1):
                        gather_blk(m2, k, bufs[p])
    for k in (3, 2, 1):
        @pl.when(total >= k)
        def _():
            wait_bytes(sem_s)
    for g in range(NG):
        buf = bufs[g % 6]
        if g >= 6:
            wait_bytes(sem_s)
        p0 = base + g * 16
        pltpu.sync_copy(t0_ref.at[pl.ds(p0, 16)], buf)
        pos_g = p0 + iota16
        b_vec = pos_g // T
        didx = b_vec * ((NR_T + 1) * T - T) + pos_g
        pltpu.async_copy(buf, out_ref.at[didx], sem_s)
    for _ in range(min(6, NG)):
        wait_bytes(sem_s)


def _tc_idx_body(noise_ref, tpm_ref, tfm_ref,
                 rem_ref, msk_ref, rev_ref, mrem_ref, mrev_ref):
    TB = T // 4
    noise = noise_ref[0]
    m_iota = jax.lax.broadcasted_iota(jnp.int32, (TB, NM), 1)
    rank = jnp.zeros((TB, NM), jnp.int32)
    for n in range(NM):
        vn = noise[:, n:n + 1]
        less = (vn < noise) | ((vn == noise) & (n < m_iota))
        rank = rank + less.astype(jnp.int32)
    rev_ref[0] = rank
    shuf = jnp.zeros((TB, NM), jnp.int32)
    for m in range(NM):
        shuf = shuf + jnp.where(rank[:, m:m + 1] == m_iota, m, 0)
    rem_ref[0] = shuf[:, :NR_T]
    msk_ref[0] = shuf[:, NR_T:]
    tpm = tpm_ref[0, :, 0]
    tfm = tfm_ref[0, :, 0]
    col5 = jax.lax.broadcasted_iota(jnp.int32, (TB, NR_T + 1), 1)
    mrem_ref[0] = jnp.where(col5 == 1, tfm[:, None], tpm[:, None])
    mrev_ref[0] = jnp.broadcast_to(tpm[:, None], (TB, NM + 1))


def _tc_img_idx_body(noise_ref, img_ref, out_ref, rem_ref, msk_ref,
                     rev_ref):
    noise = noise_ref[0, 0]
    a = noise[:, None]
    b = noise[None, :]
    ii = jax.lax.broadcasted_iota(jnp.int32, (IMG_V, IMG_V), 0)
    jj = jax.lax.broadcasted_iota(jnp.int32, (IMG_V, IMG_V), 1)
    less = (b < a) | ((b == a) & (jj < ii))
    rank = jnp.sum(less.astype(jnp.int32), axis=-1)
    rev_ref[0, 0] = rank
    rr = jax.lax.broadcasted_iota(jnp.int32, (IMG_V, IMG_V), 1)
    onehot = (rank[:, None] == rr)
    i_ids = jax.lax.broadcasted_iota(jnp.int32, (IMG_V, IMG_V), 0)
    shuf = jnp.sum(jnp.where(onehot, i_ids, 0), axis=0)
    rem_ref[0, 0] = shuf[:NR_I]
    msk_ref[0, 0] = shuf[NR_I:]
    kk = jax.lax.broadcasted_iota(jnp.int32, (IMG_V, NR_I), 1)
    oh = (rank[:, None] == kk).astype(jnp.float32)
    valid = img_ref[0, 1:, :]
    sel = jax.lax.dot_general(oh, valid, (((0,), (0,)), ((), ())),
                              precision=jax.lax.Precision.HIGHEST,
                              preferred_element_type=jnp.float32)
    out_ref[0, 0, :] = img_ref[0, 0, :]
    out_ref[0, 1:, :] = sel


def kernel(t0, t1, t2, t3, t4, t5, t6, t7, t8, img0,
           temporal_padding_mask, target_fcst_mask, noise_temporal, noise_img):
    TB = T // 4
    idx_out = pl.pallas_call(
        _tc_idx_body,
        grid=(B, 4),
        in_specs=[
            pl.BlockSpec((1, TB, NM), lambda b, t: (b, t, 0)),
            pl.BlockSpec((1, TB, 1), lambda b, t: (b, t, 0)),
            pl.BlockSpec((1, TB, 1), lambda b, t: (b, t, 0)),
        ],
        out_specs=[
            pl.BlockSpec((1, TB, NR_T), lambda b, t: (b, t, 0)),
            pl.BlockSpec((1, TB, NM - NR_T), lambda b, t: (b, t, 0)),
            pl.BlockSpec((1, TB, NM), lambda b, t: (b, t, 0)),
            pl.BlockSpec((1, TB, NR_T + 1), lambda b, t: (b, t, 0)),
            pl.BlockSpec((1, TB, NM + 1), lambda b, t: (b, t, 0)),
        ],
        out_shape=[
            jax.ShapeDtypeStruct((B, T, NR_T), jnp.int32),
            jax.ShapeDtypeStruct((B, T, NM - NR_T), jnp.int32),
            jax.ShapeDtypeStruct((B, T, NM), jnp.int32),
            jax.ShapeDtypeStruct((B, T, NR_T + 1), jnp.float32),
            jax.ShapeDtypeStruct((B, T, NM + 1), jnp.float32),
        ],
    )(noise_temporal, temporal_padding_mask[..., None], target_fcst_mask)
    remain_idx_t, masked_idx_t, revert_idx_t, t_rem_mask, t_rev_mask = idx_out

    img_out = pl.pallas_call(
        _tc_img_idx_body,
        grid=(B,),
        in_specs=[
            pl.BlockSpec((1, 1, IMG_V), lambda b: (b, 0, 0)),
            pl.BlockSpec((1, IMG_V + 1, D), lambda b: (b, 0, 0)),
        ],
        out_specs=[
            pl.BlockSpec((1, NR_I + 1, D), lambda b: (b, 0, 0)),
            pl.BlockSpec((1, 1, NR_I), lambda b: (b, 0, 0)),
            pl.BlockSpec((1, 1, IMG_V - NR_I), lambda b: (b, 0, 0)),
            pl.BlockSpec((1, 1, IMG_V), lambda b: (b, 0, 0)),
        ],
        out_shape=[
            jax.ShapeDtypeStruct((B, NR_I + 1, D), jnp.float32),
            jax.ShapeDtypeStruct((B, 1, NR_I), jnp.int32),
            jax.ShapeDtypeStruct((B, 1, IMG_V - NR_I), jnp.int32),
            jax.ShapeDtypeStruct((B, 1, IMG_V), jnp.int32),
        ],
    )(noise_img[:, None, :], img0)
    img_remain, rem3, msk3, rev3 = img_out
    remain_idx_i, masked_idx_i, revert_idx_i = rem3[:, 0], msk3[:, 0], rev3[:, 0]

    mesh = plsc.VectorSubcoreMesh(core_axis_name="c", subcore_axis_name="s")
    noise_t_tr = noise_temporal.reshape(P * NM)
    flat = lambda x: x.reshape(P, D)
    sc_temporal = functools.partial(
        pl.kernel, _sc_temporal_body, mesh=mesh,
        compiler_params=pltpu.CompilerParams(needs_layout_passes=False),
        out_type=jax.ShapeDtypeStruct((B * (NR_T + 1) * T, D), jnp.float32),
        scratch_types=[
            pltpu.VMEM((NM * C,), jnp.float32),
            pltpu.VMEM((NM * (C + 16),), jnp.int32),
            pltpu.VMEM((NM * (C + 16),), jnp.int32),
            pltpu.VMEM((16, D), jnp.float32),
            pltpu.VMEM((16, D), jnp.float32),
            pltpu.VMEM((16, D), jnp.float32),
            pltpu.VMEM((16, D), jnp.float32),
            pltpu.VMEM((16, D), jnp.float32),
            pltpu.VMEM((16, D), jnp.float32),
            pltpu.SemaphoreType.DMA,
            pltpu.SemaphoreType.DMA,
        ],
    )()
    tbr_flat = sc_temporal(noise_t_tr, flat(t0), flat(t1), flat(t2), flat(t3),
                           flat(t4), flat(t5), flat(t6), flat(t7), flat(t8))
    tbr = jnp.transpose(tbr_flat.reshape(B, NR_T + 1, T, D), (0, 2, 1, 3))

    img_rem_mask = jnp.ones((B, NR_I + 1), jnp.float32)
    img_rev_mask = jnp.ones((B, IMG_V + 1), jnp.float32)
    return (tbr, img_remain,
            t_rem_mask, t_rev_mask,
            img_rem_mask, img_rev_mask,
            remain_idx_t, masked_idx_t, revert_idx_t,
            remain_idx_i, masked_idx_i, revert_idx_i)


# speedup vs baseline: 1.0974x; 1.0974x over previous
import functools

import jax
import jax.numpy as jnp
from jax import lax
from jax.experimental import pallas as pl
from jax.experimental.pallas import tpu as pltpu
from jax.experimental.pallas import tpu_sc as plsc

B, T, D = 8, 512, 768
NM = 8
NR_T = 4
IMG_V = 576
NR_I = 144
P = B * T
NW = 32
C = P // NW
NG = C // 16
IMG_PITCH = 152


def _sc_temporal_body(noise_t_ref, t0_ref, v1, v2, v3, v4, v5, v6, v7, v8,
                      out_ref, nbuf, srcb, dstb,
                      gbufa, gbufb, gbufc, gbufd, gbufe, gbuff,
                      sem_g, sem_s):
    vals = (v1, v2, v3, v4, v5, v6, v7, v8)
    wid = lax.axis_index("s") * 2 + lax.axis_index("c")
    base = wid * C
    pltpu.sync_copy(noise_t_ref.at[pl.ds(base * NM, C * NM)], nbuf)
    iota16 = lax.broadcasted_iota(jnp.int32, (16,), 0)
    LROW = C + 16
    offs = [0] * NM
    last_pair = [jnp.full((16,), -1, jnp.int32) for _ in range(NM)]
    for g in range(NG):
        lane_base = (g * 16 + iota16) * NM
        nvs = [plsc.load_gather(nbuf, [lane_base + m]) for m in range(NM)]
        pos = base + g * 16 + iota16
        for m in range(NM):
            r = jnp.zeros((16,), jnp.int32)
            for n in range(NM):
                if n == m:
                    continue
                cond = nvs[n] < nvs[m]
                if n < m:
                    cond = cond | (nvs[n] == nvs[m])
                r = r + cond.astype(jnp.int32)
            mask = r < NR_T
            b_vec = pos // T
            t_vec = pos - b_vec * T
            dst = (b_vec * (NR_T + 1) + 1 + r) * T + t_vec
            cs = plsc.cumsum(mask.astype(jnp.int32))
            slot = m * LROW + offs[m] + cs - 1
            plsc.store_scatter(srcb, [slot], pos, mask=mask)
            plsc.store_scatter(dstb, [slot], dst, mask=mask)
            offs[m] = offs[m] + jnp.max(
                plsc.all_reduce_population_count(mask))
            last_pair[m] = jnp.maximum(
                last_pair[m], jnp.where(mask, pos * 32768 + dst, -1))
    zero16 = jnp.zeros((16,), jnp.int32)
    for m in range(NM):
        tail = m * LROW + offs[m] + iota16
        best = jnp.max(last_pair[m])
        ls = best // 32768
        plsc.store_scatter(srcb, [tail], zero16 + ls, mask=iota16 >= 0)
        plsc.store_scatter(dstb, [tail], zero16 + (best - ls * 32768),
                           mask=iota16 >= 0)
    bufs = (gbufa, gbufb, gbufc, gbufd, gbufe, gbuff)

    def wait_bytes(sem):
        pltpu.make_async_copy(t0_ref.at[pl.ds(0, 16)], gbufa, sem).wait()

    def gather_blk(m, i, buf):
        sidx = srcb[pl.ds(m * LROW + i * 16, 16)]
        pltpu.async_copy(vals[m].at[sidx], buf, sem_g)

    def scatter_blk(m, i, buf):
        didx = dstb[pl.ds(m * LROW + i * 16, 16)]
        pltpu.async_copy(buf, out_ref.at[didx], sem_s)

    nbs = [(offs[m] + 15) // 16 for m in range(NM)]
    for k in range(3):
        @pl.when(nbs[0] > k)
        def _(k=k):
            gather_blk(0, k, bufs[k])
    total = 0
    for m in range(NM):
        nb = nbs[m]

        def it(i, carry, m=m, goff=total, nb=nb):
            g = goff + i
            @pl.when(g >= 3)
            def _():
                wait_bytes(sem_s)
            for p in range(6):
                @pl.when(((g + 3) % 6 == p) & (i + 3 < nb))
                def _(p=p):
                    gather_blk(m, i + 3, bufs[p])
            wait_bytes(sem_g)
            for p in range(6):
                @pl.when(g % 6 == p)
                def _(p=p):
                    scatter_blk(m, i, bufs[p])
            return carry

        lax.fori_loop(0, nb, it, 0)
        total = total + nb
        if m + 1 < NM:
            for k in range(3):
                for p in range(6):
                    @pl.when((nbs[m + 1] > k) & ((total + k) % 6 == p))
                    def _(k=k, p=p, m2=m + 1):
                        gather_blk(m2, k, bufs[p])
    for k in (3, 2, 1):
        @pl.when(total >= k)
        def _():
            wait_bytes(sem_s)
    for g in range(NG):
        buf = bufs[g % 6]
        if g >= 6:
            wait_bytes(sem_s)
        p0 = base + g * 16
        pltpu.sync_copy(t0_ref.at[pl.ds(p0, 16)], buf)
        pos_g = p0 + iota16
        b_vec = pos_g // T
        didx = b_vec * ((NR_T + 1) * T - T) + pos_g
        pltpu.async_copy(buf, out_ref.at[didx], sem_s)
    for _ in range(min(6, NG)):
        wait_bytes(sem_s)


def _tc_idx_body(noise_ref, tpm_ref, tfm_ref,
                 rem_ref, msk_ref, rev_ref, mrem_ref, mrev_ref):
    TB = T // 4
    noise = noise_ref[0]
    m_iota = jax.lax.broadcasted_iota(jnp.int32, (TB, NM), 1)
    rank = jnp.zeros((TB, NM), jnp.int32)
    for n in range(NM):
        vn = noise[:, n:n + 1]
        less = (vn < noise) | ((vn == noise) & (n < m_iota))
        rank = rank + less.astype(jnp.int32)
    rev_ref[0] = rank
    shuf = jnp.zeros((TB, NM), jnp.int32)
    for m in range(NM):
        shuf = shuf + jnp.where(rank[:, m:m + 1] == m_iota, m, 0)
    rem_ref[0] = shuf[:, :NR_T]
    msk_ref[0] = shuf[:, NR_T:]
    tpm = tpm_ref[0, :, 0]
    tfm = tfm_ref[0, :, 0]
    col5 = jax.lax.broadcasted_iota(jnp.int32, (TB, NR_T + 1), 1)
    mrem_ref[0] = jnp.where(col5 == 1, tfm[:, None], tpm[:, None])
    mrev_ref[0] = jnp.broadcast_to(tpm[:, None], (TB, NM + 1))


def _tc_img_idx_body(noise_ref, rem_ref, msk_ref, rev_ref):
    noise = noise_ref[0, 0]
    a = noise[:, None]
    b = noise[None, :]
    ii = jax.lax.broadcasted_iota(jnp.int32, (IMG_V, IMG_V), 0)
    jj = jax.lax.broadcasted_iota(jnp.int32, (IMG_V, IMG_V), 1)
    less = (b < a) | ((b == a) & (jj < ii))
    rank = jnp.sum(less.astype(jnp.int32), axis=-1)
    rev_ref[0, 0] = rank
    rr = jax.lax.broadcasted_iota(jnp.int32, (IMG_V, IMG_V), 1)
    onehot = (rank[:, None] == rr)
    i_ids = jax.lax.broadcasted_iota(jnp.int32, (IMG_V, IMG_V), 0)
    shuf = jnp.sum(jnp.where(onehot, i_ids, 0), axis=0)
    rem_ref[0, 0] = shuf[:NR_I]
    msk_ref[0, 0] = shuf[NR_I:]


def _sc_img_body(img_ref, ridx_ref, out_ref, idxv, gbuf, sem_in, sem_out):
    wid = lax.axis_index("s") * 2 + lax.axis_index("c")
    iota16 = lax.broadcasted_iota(jnp.int32, (16,), 0)
    nblk = NR_I // 16
    njob = B * nblk + B
    for t in range((njob + NW - 1) // NW):
        job = wid + t * NW

        @pl.when(job < B * nblk)
        def _():
            b = job // nblk
            k = job % nblk
            pltpu.sync_copy(ridx_ref.at[pl.ds(b * NR_I + k * 16, 16)], idxv)
            src = idxv[...] + (b * (IMG_V + 1) + 1)
            pltpu.async_copy(img_ref.at[src], gbuf, sem_in).wait()
            didx = (1 + k * 16 + iota16) * B + b
            pltpu.async_copy(gbuf, out_ref.at[didx], sem_out).wait()

        @pl.when((job >= B * nblk) & (job < njob))
        def _():
            b = job - B * nblk
            src = jnp.zeros((16,), jnp.int32) + b * (IMG_V + 1)
            pltpu.async_copy(img_ref.at[src], gbuf, sem_in).wait()
            didx = jnp.zeros((16,), jnp.int32) + b
            pltpu.async_copy(gbuf, out_ref.at[didx], sem_out).wait()


def kernel(t0, t1, t2, t3, t4, t5, t6, t7, t8, img0,
           temporal_padding_mask, target_fcst_mask, noise_temporal, noise_img):
    TB = T // 4
    idx_out = pl.pallas_call(
        _tc_idx_body,
        grid=(B, 4),
        in_specs=[
            pl.BlockSpec((1, TB, NM), lambda b, t: (b, t, 0)),
            pl.BlockSpec((1, TB, 1), lambda b, t: (b, t, 0)),
            pl.BlockSpec((1, TB, 1), lambda b, t: (b, t, 0)),
        ],
        out_specs=[
            pl.BlockSpec((1, TB, NR_T), lambda b, t: (b, t, 0)),
            pl.BlockSpec((1, TB, NM - NR_T), lambda b, t: (b, t, 0)),
            pl.BlockSpec((1, TB, NM), lambda b, t: (b, t, 0)),
            pl.BlockSpec((1, TB, NR_T + 1), lambda b, t: (b, t, 0)),
            pl.BlockSpec((1, TB, NM + 1), lambda b, t: (b, t, 0)),
        ],
        out_shape=[
            jax.ShapeDtypeStruct((B, T, NR_T), jnp.int32),
            jax.ShapeDtypeStruct((B, T, NM - NR_T), jnp.int32),
            jax.ShapeDtypeStruct((B, T, NM), jnp.int32),
            jax.ShapeDtypeStruct((B, T, NR_T + 1), jnp.float32),
            jax.ShapeDtypeStruct((B, T, NM + 1), jnp.float32),
        ],
    )(noise_temporal, temporal_padding_mask[..., None], target_fcst_mask)
    remain_idx_t, masked_idx_t, revert_idx_t, t_rem_mask, t_rev_mask = idx_out

    img_idx = pl.pallas_call(
        _tc_img_idx_body,
        grid=(B,),
        in_specs=[pl.BlockSpec((1, 1, IMG_V), lambda b: (b, 0, 0))],
        out_specs=[
            pl.BlockSpec((1, 1, NR_I), lambda b: (b, 0, 0)),
            pl.BlockSpec((1, 1, IMG_V - NR_I), lambda b: (b, 0, 0)),
            pl.BlockSpec((1, 1, IMG_V), lambda b: (b, 0, 0)),
        ],
        out_shape=[
            jax.ShapeDtypeStruct((B, 1, NR_I), jnp.int32),
            jax.ShapeDtypeStruct((B, 1, IMG_V - NR_I), jnp.int32),
            jax.ShapeDtypeStruct((B, 1, IMG_V), jnp.int32),
        ],
    )(noise_img[:, None, :])
    remain_idx_i, masked_idx_i, revert_idx_i = (o[:, 0] for o in img_idx)

    mesh = plsc.VectorSubcoreMesh(core_axis_name="c", subcore_axis_name="s")
    noise_t_tr = noise_temporal.reshape(P * NM)
    flat = lambda x: x.reshape(P, D)
    sc_temporal = functools.partial(
        pl.kernel, _sc_temporal_body, mesh=mesh,
        compiler_params=pltpu.CompilerParams(needs_layout_passes=False),
        out_type=jax.ShapeDtypeStruct((B * (NR_T + 1) * T, D), jnp.float32),
        scratch_types=[
            pltpu.VMEM((NM * C,), jnp.float32),
            pltpu.VMEM((NM * (C + 16),), jnp.int32),
            pltpu.VMEM((NM * (C + 16),), jnp.int32),
            pltpu.VMEM((16, D), jnp.float32),
            pltpu.VMEM((16, D), jnp.float32),
            pltpu.VMEM((16, D), jnp.float32),
            pltpu.VMEM((16, D), jnp.float32),
            pltpu.VMEM((16, D), jnp.float32),
            pltpu.VMEM((16, D), jnp.float32),
            pltpu.SemaphoreType.DMA,
            pltpu.SemaphoreType.DMA,
        ],
    )()
    tbr_flat = sc_temporal(noise_t_tr, flat(t0), flat(t1), flat(t2), flat(t3),
                           flat(t4), flat(t5), flat(t6), flat(t7), flat(t8))
    tbr = jnp.transpose(tbr_flat.reshape(B, NR_T + 1, T, D), (0, 2, 1, 3))

    sc_img = functools.partial(
        pl.kernel, _sc_img_body, mesh=mesh,
        compiler_params=pltpu.CompilerParams(needs_layout_passes=False),
        out_type=jax.ShapeDtypeStruct(((NR_I + 1) * B, D), jnp.float32),
        scratch_types=[
            pltpu.VMEM((16,), jnp.int32),
            pltpu.VMEM((16, D), jnp.float32),
            pltpu.SemaphoreType.DMA,
            pltpu.SemaphoreType.DMA,
        ],
    )()
    img_flat = sc_img(img0.reshape(B * (IMG_V + 1), D),
                      remain_idx_i.reshape(B * NR_I))
    img_remain = jnp.transpose(img_flat.reshape(NR_I + 1, B, D), (1, 0, 2))

    img_rem_mask = jnp.ones((B, NR_I + 1), jnp.float32)
    img_rev_mask = jnp.ones((B, IMG_V + 1), jnp.float32)
    return (tbr, img_remain,
            t_rem_mask, t_rev_mask,
            img_rem_mask, img_rev_mask,
            remain_idx_t, masked_idx_t, revert_idx_t,
            remain_idx_i, masked_idx_i, revert_idx_i)


# speedup vs baseline: 1.0991x; 1.0016x over previous
import functools

import jax
import jax.numpy as jnp
from jax import lax
from jax.experimental import pallas as pl
from jax.experimental.pallas import tpu as pltpu
from jax.experimental.pallas import tpu_sc as plsc

B, T, D = 8, 512, 768
NM = 8
NR_T = 4
IMG_V = 576
NR_I = 144
P = B * T
NW = 32
C = P // NW
NG = C // 16
IMG_PITCH = 152


def _sc_temporal_body(noise_t_ref, t0_ref, v1, v2, v3, v4, v5, v6, v7, v8,
                      out_ref, nbuf, srcb, dstb,
                      gbufa, gbufb, gbufc, gbufd, gbufe, gbuff,
                      sem_g, sem_s):
    vals = (v1, v2, v3, v4, v5, v6, v7, v8)
    wid = lax.axis_index("s") * 2 + lax.axis_index("c")
    base = wid * C
    pltpu.sync_copy(noise_t_ref.at[pl.ds(base * NM, C * NM)], nbuf)
    iota16 = lax.broadcasted_iota(jnp.int32, (16,), 0)
    LROW = C + 16
    offs = [0] * NM
    last_pair = [jnp.full((16,), -1, jnp.int32) for _ in range(NM)]
    for g in range(NG):
        lane_base = (g * 16 + iota16) * NM
        nvs = [plsc.load_gather(nbuf, [lane_base + m]) for m in range(NM)]
        pos = base + g * 16 + iota16
        for m in range(NM):
            r = jnp.zeros((16,), jnp.int32)
            for n in range(NM):
                if n == m:
                    continue
                cond = nvs[n] < nvs[m]
                if n < m:
                    cond = cond | (nvs[n] == nvs[m])
                r = r + cond.astype(jnp.int32)
            mask = r < NR_T
            b_vec = pos // T
            t_vec = pos - b_vec * T
            dst = (b_vec * (NR_T + 1) + 1 + r) * T + t_vec
            cs = plsc.cumsum(mask.astype(jnp.int32))
            slot = m * LROW + offs[m] + cs - 1
            plsc.store_scatter(srcb, [slot], pos, mask=mask)
            plsc.store_scatter(dstb, [slot], dst, mask=mask)
            offs[m] = offs[m] + jnp.max(
                plsc.all_reduce_population_count(mask))
            last_pair[m] = jnp.maximum(
                last_pair[m], jnp.where(mask, pos * 32768 + dst, -1))
    zero16 = jnp.zeros((16,), jnp.int32)
    for m in range(NM):
        tail = m * LROW + offs[m] + iota16
        best = jnp.max(last_pair[m])
        ls = best // 32768
        plsc.store_scatter(srcb, [tail], zero16 + ls, mask=iota16 >= 0)
        plsc.store_scatter(dstb, [tail], zero16 + (best - ls * 32768),
                           mask=iota16 >= 0)
    bufs = (gbufa, gbufb, gbufc, gbufd, gbufe, gbuff)

    def wait_bytes(sem):
        pltpu.make_async_copy(t0_ref.at[pl.ds(0, 16)], gbufa, sem).wait()

    def gather_blk(m, i, buf):
        sidx = srcb[pl.ds(m * LROW + i * 16, 16)]
        pltpu.async_copy(vals[m].at[sidx], buf, sem_g)

    def scatter_blk(m, i, buf):
        didx = dstb[pl.ds(m * LROW + i * 16, 16)]
        pltpu.async_copy(buf, out_ref.at[didx], sem_s)

    nbs = [(offs[m] + 15) // 16 for m in range(NM)]
    for k in range(3):
        @pl.when(nbs[0] > k)
        def _(k=k):
            gather_blk(0, k, bufs[k])
    total = 0
    for m in range(NM):
        nb = nbs[m]

        def it(i, carry, m=m, goff=total, nb=nb):
            g = goff + i
            @pl.when(g >= 3)
            def _():
                wait_bytes(sem_s)
            for p in range(6):
                @pl.when(((g + 3) % 6 == p) & (i + 3 < nb))
                def _(p=p):
                    gather_blk(m, i + 3, bufs[p])
            wait_bytes(sem_g)
            for p in range(6):
                @pl.when(g % 6 == p)
                def _(p=p):
                    scatter_blk(m, i, bufs[p])
            return carry

        lax.fori_loop(0, nb, it, 0)
        total = total + nb
        if m + 1 < NM:
            for k in range(3):
                for p in range(6):
                    @pl.when((nbs[m + 1] > k) & ((total + k) % 6 == p))
                    def _(k=k, p=p, m2=m + 1):
                        gather_blk(m2, k, bufs[p])
    for k in (3, 2, 1):
        @pl.when(total >= k)
        def _():
            wait_bytes(sem_s)
    for g in range(NG):
        buf = bufs[g % 6]
        if g >= 6:
            wait_bytes(sem_s)
        p0 = base + g * 16
        pltpu.sync_copy(t0_ref.at[pl.ds(p0, 16)], buf)
        pos_g = p0 + iota16
        b_vec = pos_g // T
        didx = b_vec * ((NR_T + 1) * T - T) + pos_g
        pltpu.async_copy(buf, out_ref.at[didx], sem_s)
    for _ in range(min(6, NG)):
        wait_bytes(sem_s)


def _tc_idx_body(noise_ref, tpm_ref, tfm_ref,
                 rem_ref, msk_ref, rev_ref, mrem_ref, mrev_ref):
    TB = T // 4
    noise = noise_ref[0]
    m_iota = jax.lax.broadcasted_iota(jnp.int32, (TB, NM), 1)
    rank = jnp.zeros((TB, NM), jnp.int32)
    for n in range(NM):
        vn = noise[:, n:n + 1]
        less = (vn < noise) | ((vn == noise) & (n < m_iota))
        rank = rank + less.astype(jnp.int32)
    rev_ref[0] = rank
    shuf = jnp.zeros((TB, NM), jnp.int32)
    for m in range(NM):
        shuf = shuf + jnp.where(rank[:, m:m + 1] == m_iota, m, 0)
    rem_ref[0] = shuf[:, :NR_T]
    msk_ref[0] = shuf[:, NR_T:]
    tpm = tpm_ref[0, :, 0]
    tfm = tfm_ref[0, :, 0]
    col5 = jax.lax.broadcasted_iota(jnp.int32, (TB, NR_T + 1), 1)
    mrem_ref[0] = jnp.where(col5 == 1, tfm[:, None], tpm[:, None])
    mrev_ref[0] = jnp.broadcast_to(tpm[:, None], (TB, NM + 1))


def _tc_img_idx_body(noise_ref, rem_ref, msk_ref, rev_ref, rem256_ref):
    noise = noise_ref[0, 0]
    a = noise[:, None]
    b = noise[None, :]
    ii = jax.lax.broadcasted_iota(jnp.int32, (IMG_V, IMG_V), 0)
    jj = jax.lax.broadcasted_iota(jnp.int32, (IMG_V, IMG_V), 1)
    less = (b < a) | ((b == a) & (jj < ii))
    rank = jnp.sum(less.astype(jnp.int32), axis=-1)
    rev_ref[0, 0] = rank
    rr = jax.lax.broadcasted_iota(jnp.int32, (IMG_V, IMG_V), 1)
    onehot = (rank[:, None] == rr)
    i_ids = jax.lax.broadcasted_iota(jnp.int32, (IMG_V, IMG_V), 0)
    shuf = jnp.sum(jnp.where(onehot, i_ids, 0), axis=0)
    rem_ref[0, 0] = shuf[:NR_I]
    msk_ref[0, 0] = shuf[NR_I:]
    rem256_ref[0, 0] = jnp.concatenate(
        [shuf[:NR_I], jnp.zeros((256 - NR_I,), jnp.int32)])


def _sc_img_body(img_ref, ridx_ref, out_ref, idxv, gbuf, sem_in, sem_out):
    wid = lax.axis_index("s") * 2 + lax.axis_index("c")
    iota16 = lax.broadcasted_iota(jnp.int32, (16,), 0)
    nblk = NR_I // 16
    njob = B * nblk + B
    for t in range((njob + NW - 1) // NW):
        job = wid + t * NW

        @pl.when(job < B * nblk)
        def _():
            b = job // nblk
            k = job % nblk
            pltpu.sync_copy(ridx_ref.at[pl.ds(b * 256 + k * 16, 16)], idxv)
            src = idxv[...] + (b * (IMG_V + 1) + 1)
            pltpu.async_copy(img_ref.at[src], gbuf, sem_in).wait()
            didx = (1 + k * 16 + iota16) * B + b
            pltpu.async_copy(gbuf, out_ref.at[didx], sem_out).wait()

        @pl.when((job >= B * nblk) & (job < njob))
        def _():
            b = job - B * nblk
            src = jnp.zeros((16,), jnp.int32) + b * (IMG_V + 1)
            pltpu.async_copy(img_ref.at[src], gbuf, sem_in).wait()
            didx = jnp.zeros((16,), jnp.int32) + b
            pltpu.async_copy(gbuf, out_ref.at[didx], sem_out).wait()


def kernel(t0, t1, t2, t3, t4, t5, t6, t7, t8, img0,
           temporal_padding_mask, target_fcst_mask, noise_temporal, noise_img):
    TB = T // 4
    idx_out = pl.pallas_call(
        _tc_idx_body,
        grid=(B, 4),
        in_specs=[
            pl.BlockSpec((1, TB, NM), lambda b, t: (b, t, 0)),
            pl.BlockSpec((1, TB, 1), lambda b, t: (b, t, 0)),
            pl.BlockSpec((1, TB, 1), lambda b, t: (b, t, 0)),
        ],
        out_specs=[
            pl.BlockSpec((1, TB, NR_T), lambda b, t: (b, t, 0)),
            pl.BlockSpec((1, TB, NM - NR_T), lambda b, t: (b, t, 0)),
            pl.BlockSpec((1, TB, NM), lambda b, t: (b, t, 0)),
            pl.BlockSpec((1, TB, NR_T + 1), lambda b, t: (b, t, 0)),
            pl.BlockSpec((1, TB, NM + 1), lambda b, t: (b, t, 0)),
        ],
        out_shape=[
            jax.ShapeDtypeStruct((B, T, NR_T), jnp.int32),
            jax.ShapeDtypeStruct((B, T, NM - NR_T), jnp.int32),
            jax.ShapeDtypeStruct((B, T, NM), jnp.int32),
            jax.ShapeDtypeStruct((B, T, NR_T + 1), jnp.float32),
            jax.ShapeDtypeStruct((B, T, NM + 1), jnp.float32),
        ],
    )(noise_temporal, temporal_padding_mask[..., None], target_fcst_mask)
    remain_idx_t, masked_idx_t, revert_idx_t, t_rem_mask, t_rev_mask = idx_out

    img_idx = pl.pallas_call(
        _tc_img_idx_body,
        grid=(B,),
        in_specs=[pl.BlockSpec((1, 1, IMG_V), lambda b: (b, 0, 0))],
        out_specs=[
            pl.BlockSpec((1, 1, NR_I), lambda b: (b, 0, 0)),
            pl.BlockSpec((1, 1, IMG_V - NR_I), lambda b: (b, 0, 0)),
            pl.BlockSpec((1, 1, IMG_V), lambda b: (b, 0, 0)),
            pl.BlockSpec((1, 1, 256), lambda b: (b, 0, 0)),
        ],
        out_shape=[
            jax.ShapeDtypeStruct((B, 1, NR_I), jnp.int32),
            jax.ShapeDtypeStruct((B, 1, IMG_V - NR_I), jnp.int32),
            jax.ShapeDtypeStruct((B, 1, IMG_V), jnp.int32),
            jax.ShapeDtypeStruct((B, 1, 256), jnp.int32),
        ],
    )(noise_img[:, None, :])
    remain_idx_i, masked_idx_i, revert_idx_i, rem256 = (
        o[:, 0] for o in img_idx)

    mesh = plsc.VectorSubcoreMesh(core_axis_name="c", subcore_axis_name="s")
    noise_t_tr = noise_temporal.reshape(P * NM)
    flat = lambda x: x.reshape(P, D)
    sc_temporal = functools.partial(
        pl.kernel, _sc_temporal_body, mesh=mesh,
        compiler_params=pltpu.CompilerParams(needs_layout_passes=False),
        out_type=jax.ShapeDtypeStruct((B * (NR_T + 1) * T, D), jnp.float32),
        scratch_types=[
            pltpu.VMEM((NM * C,), jnp.float32),
            pltpu.VMEM((NM * (C + 16),), jnp.int32),
            pltpu.VMEM((NM * (C + 16),), jnp.int32),
            pltpu.VMEM((16, D), jnp.float32),
            pltpu.VMEM((16, D), jnp.float32),
            pltpu.VMEM((16, D), jnp.float32),
            pltpu.VMEM((16, D), jnp.float32),
            pltpu.VMEM((16, D), jnp.float32),
            pltpu.VMEM((16, D), jnp.float32),
            pltpu.SemaphoreType.DMA,
            pltpu.SemaphoreType.DMA,
        ],
    )()
    tbr_flat = sc_temporal(noise_t_tr, flat(t0), flat(t1), flat(t2), flat(t3),
                           flat(t4), flat(t5), flat(t6), flat(t7), flat(t8))
    tbr = jnp.transpose(tbr_flat.reshape(B, NR_T + 1, T, D), (0, 2, 1, 3))

    sc_img = functools.partial(
        pl.kernel, _sc_img_body, mesh=mesh,
        compiler_params=pltpu.CompilerParams(needs_layout_passes=False),
        out_type=jax.ShapeDtypeStruct(((NR_I + 1) * B, D), jnp.float32),
        scratch_types=[
            pltpu.VMEM((16,), jnp.int32),
            pltpu.VMEM((16, D), jnp.float32),
            pltpu.SemaphoreType.DMA,
            pltpu.SemaphoreType.DMA,
        ],
    )()
    img_flat = sc_img(img0.reshape(B * (IMG_V + 1), D),
                      rem256.reshape(B * 256))
    img_remain = jnp.transpose(img_flat.reshape(NR_I + 1, B, D), (1, 0, 2))

    img_rem_mask = jnp.ones((B, NR_I + 1), jnp.float32)
    img_rev_mask = jnp.ones((B, IMG_V + 1), jnp.float32)
    return (tbr, img_remain,
            t_rem_mask, t_rev_mask,
            img_rem_mask, img_rev_mask,
            remain_idx_t, masked_idx_t, revert_idx_t,
            remain_idx_i, masked_idx_i, revert_idx_i)
